# baseline (speedup 1.0000x reference)
import functools

import jax
import jax.numpy as jnp
from jax import lax
from jax.experimental import pallas as pl
from jax.experimental.pallas import tpu as pltpu
from jax.experimental.pallas import tpu_sc as plsc

E = 16
D = 2048
D_FF = 1024
NTOK = 2048
TB = 256
NBLK = 16
NSLOT = NBLK * TB
CS = D // 2


def _bc16(s):
    return jnp.full((16,), s, jnp.int32)


def _router_body(x_ref, wg_ref, logits_ref, ep_ref, w0_ref, w1_ref):
    x = x_ref[...]
    logits = jax.lax.dot_general(
        x, wg_ref[...], (((1,), (1,)), ((), ())),
        preferred_element_type=jnp.float32,
    )
    logits_ref[...] = logits

    tb = logits.shape[0]
    lane = jax.lax.broadcasted_iota(jnp.int32, (tb, E), 1)
    neg = jnp.float32(-jnp.inf)

    gsum, gmax, garg = [], [], []
    for g in range(4):
        mask = (lane // 4) == g
        gsum.append(jnp.sum(jnp.where(mask, logits, 0.0), axis=1, keepdims=True))
        mg = jnp.max(jnp.where(mask, logits, neg), axis=1, keepdims=True)
        gmax.append(mg)
        ag = jnp.min(
            jnp.where(mask & (logits == mg), lane, 99), axis=1, keepdims=True
        ) - 4 * g
        garg.append(ag)

    ew, probs = [], []
    for s in range(2):
        pick = gsum[2 * s + 1] > gsum[2 * s]
        e_s = jnp.where(pick, 4 + garg[2 * s + 1], garg[2 * s])
        p_s = jnp.where(pick, gmax[2 * s + 1], gmax[2 * s])
        ew.append(e_s)
        probs.append(p_s)

    m = jnp.maximum(probs[0], probs[1])
    z0 = jnp.exp(probs[0] - m)
    z1 = jnp.exp(probs[1] - m)
    w0_ref[...] = z0 / (z0 + z1)
    w1_ref[...] = z1 / (z0 + z1)
    ep_ref[...] = ew[0] + 8 * ew[1]


def _router(x, Wgate):
    tb = 256
    return pl.pallas_call(
        _router_body,
        grid=(NTOK // tb,),
        in_specs=[
            pl.BlockSpec((tb, D), lambda i: (i, 0)),
            pl.BlockSpec((E, D), lambda i: (0, 0)),
        ],
        out_specs=[
            pl.BlockSpec((tb, E), lambda i: (i, 0)),
            pl.BlockSpec((tb, 1), lambda i: (i, 0)),
            pl.BlockSpec((tb, 1), lambda i: (i, 0)),
            pl.BlockSpec((tb, 1), lambda i: (i, 0)),
        ],
        out_shape=[
            jax.ShapeDtypeStruct((NTOK, E), jnp.float32),
            jax.ShapeDtypeStruct((NTOK, 1), jnp.int32),
            jax.ShapeDtypeStruct((NTOK, 1), jnp.float32),
            jax.ShapeDtypeStruct((NTOK, 1), jnp.float32),
        ],
    )(x, Wgate)


def _sort_gather(x, ep, wcat):
    mesh = plsc.VectorSubcoreMesh(core_axis_name="c", subcore_axis_name="s")

    @functools.partial(
        pl.kernel,
        mesh=mesh,
        compiler_params=pltpu.CompilerParams(needs_layout_passes=False),
        out_type=[
            jax.ShapeDtypeStruct((2 * NSLOT, D), jnp.float32),
            jax.ShapeDtypeStruct((2 * NTOK,), jnp.int32),
            jax.ShapeDtypeStruct((2 * NSLOT,), jnp.float32),
            jax.ShapeDtypeStruct((2 * NBLK,), jnp.int32),
        ],
        scratch_types=[
            pltpu.VMEM((128,), jnp.int32),
            pltpu.VMEM((128,), jnp.float32),
            pltpu.VMEM((128,), jnp.int32),
            pltpu.VMEM((128,), jnp.int32),
            pltpu.VMEM((128,), jnp.int32),
            pltpu.VMEM((16,), jnp.int32),
            pltpu.VMEM((16, 16), jnp.int32),
            pltpu.VMEM((16,), jnp.int32),
            pltpu.VMEM((16,), jnp.int32),
            pltpu.VMEM((16,), jnp.int32),
            pltpu.VMEM((256,), jnp.int32),
            pltpu.VMEM((256,), jnp.float32),
            pltpu.VMEM((16, D), jnp.float32),
            pltpu.VMEM((16, D), jnp.float32),
            pltpu.VMEM_SHARED((16, 16), jnp.int32),
            pltpu.VMEM_SHARED((NSLOT,), jnp.int32),
            pltpu.VMEM_SHARED((NSLOT,), jnp.float32),
            pltpu.SemaphoreType.DMA,
            pltpu.SemaphoreType.DMA,
        ],
    )
    def k(x_hbm, ep_hbm, wcat_hbm, xs_hbm, t2s_hbm, wsrt_hbm, beid_hbm,
          e_v, w_v, rank_v, slots_v, g_v, counts_v, allc_v, base_v, bo_v,
          beid_v, idx_v, wbuf_v, rows_v, rows2_v, counts_sh, sidx_sh,
          wsrt_sh, sem, sem2):
        c = lax.axis_index("c")
        t = lax.axis_index("s")
        tok0 = t * 128
        iota = lax.iota(jnp.int32, 16)

        pltpu.sync_copy(ep_hbm.at[pl.ds(tok0, 128)], e_v)
        pltpu.sync_copy(wcat_hbm.at[pl.ds(c * NTOK + tok0, 128)], w_v)
        c_is_1 = _bc16(c) == 1
        for i in range(8):
            v = e_v[pl.ds(16 * i, 16)]
            e_v[pl.ds(16 * i, 16)] = jnp.where(c_is_1, (v >> 3) & 7, v & 7)

        counts_v[...] = jnp.zeros((16,), jnp.int32)
        for i in range(8):
            ev = e_v[pl.ds(16 * i, 16)]
            prefix = plsc.load_gather(counts_v, [ev])
            within = jnp.zeros((16,), jnp.int32)
            newc = counts_v[...]
            for e in range(8):
                m = ev == e
                mi = m.astype(jnp.int32)
                cum = jnp.cumsum(mi)
                within = jnp.where(m, cum - 1, within)
                newc = newc + jnp.where(iota == e, _bc16(jnp.sum(mi)), 0)
            counts_v[...] = newc
            rank_v[pl.ds(16 * i, 16)] = prefix + within

        pltpu.sync_copy(counts_v, counts_sh.at[t])
        plsc.subcore_barrier()
        pltpu.sync_copy(counts_sh, allc_v)
        ctot = allc_v[0]
        for i in range(1, 16):
            ctot = ctot + allc_v[i]
        pre = jnp.zeros((16,), jnp.int32)
        tvec = _bc16(t)
        for i in range(16):
            pre = pre + jnp.where(tvec > i, allc_v[i], 0)
        padded = ((ctot + (TB - 1)) >> 8) << 8
        poff = jnp.cumsum(padded) - padded
        base_v[...] = poff + pre
        bo_v[...] = poff >> 8

        @pl.when(t == 0)
        def _():
            beid = jnp.zeros((16,), jnp.int32)
            for e in range(1, 8):
                boe = plsc.load_gather(bo_v, [_bc16(e)])
                beid = beid + jnp.where(iota >= boe, 1, 0)
            beid_v[...] = beid + _bc16(8 * c)
            pltpu.sync_copy(beid_v, beid_hbm.at[pl.ds(NBLK * c, NBLK)])

        for i in range(8):
            ev = e_v[pl.ds(16 * i, 16)]
            slots = plsc.load_gather(base_v, [ev]) + rank_v[pl.ds(16 * i, 16)]
            slots_v[pl.ds(16 * i, 16)] = slots
            g_v[pl.ds(16 * i, 16)] = slots + _bc16(c * NSLOT)
            e_v[pl.ds(16 * i, 16)] = iota + _bc16(tok0 + 16 * i)
        pltpu.sync_copy(g_v, t2s_hbm.at[pl.ds(c * NTOK + tok0, 128)])

        for j in range(16):
            idx_v[pl.ds(16 * j, 16)] = jnp.zeros((16,), jnp.int32)
        pltpu.sync_copy(idx_v, sidx_sh.at[pl.ds(256 * t, 256)])
        plsc.subcore_barrier()
        pltpu.sync_copy(e_v, sidx_sh.at[slots_v])
        pltpu.sync_copy(w_v, wsrt_sh.at[slots_v])
        plsc.subcore_barrier()

        pltpu.sync_copy(wsrt_sh.at[pl.ds(256 * t, 256)], wbuf_v)
        pltpu.sync_copy(wbuf_v, wsrt_hbm.at[pl.ds(c * NSLOT + 256 * t, 256)])
        pltpu.sync_copy(sidx_sh.at[pl.ds(256 * t, 256)], idx_v)
        bufs = (rows_v, rows2_v)
        sems = (sem, sem2)
        handles = [None] * 16

        def _issue(j):
            return pltpu.async_copy(
                x_hbm.at[idx_v.at[pl.ds(16 * j, 16)]], bufs[j % 2], sems[j % 2]
            )

        handles[0] = _issue(0)
        for j in range(16):
            if j + 1 < 16:
                handles[j + 1] = _issue(j + 1)
            handles[j].wait()
            pltpu.sync_copy(
                bufs[j % 2], xs_hbm.at[pl.ds(c * NSLOT + 256 * t + 16 * j, 16)]
            )

    return k(x, ep, wcat)


def _ffn_body(beid_ref, xs_ref, wgc_ref, wuc_ref, wdc_ref, ws_ref, out_ref):
    kf = pl.program_id(1)

    @pl.when(kf == 0)
    def _():
        out_ref[...] = jnp.zeros_like(out_ref)

    x = xs_ref[...]
    g = jax.lax.dot_general(x, wgc_ref[0], (((1,), (1,)), ((), ())),
                            preferred_element_type=jnp.float32)
    u = jax.lax.dot_general(x, wuc_ref[0], (((1,), (1,)), ((), ())),
                            preferred_element_type=jnp.float32)
    hc = (g * jax.nn.sigmoid(g)) * u
    z = jax.lax.dot_general(hc, wdc_ref[0], (((1,), (1,)), ((), ())),
                            preferred_element_type=jnp.float32)
    out_ref[...] += ws_ref[...] * z


def _ffn_grouped(beid, xs, Wg, Wu, Wd, ws):
    ffc = 512
    grid_spec = pltpu.PrefetchScalarGridSpec(
        num_scalar_prefetch=1,
        grid=(2 * NBLK, D_FF // ffc),
        in_specs=[
            pl.BlockSpec((TB, D), lambda i, k, b: (i, 0)),
            pl.BlockSpec((1, ffc, D), lambda i, k, b: (b[i], k, 0)),
            pl.BlockSpec((1, ffc, D), lambda i, k, b: (b[i], k, 0)),
            pl.BlockSpec((1, CS, ffc), lambda i, k, b: (b[i], 0, k)),
            pl.BlockSpec((TB, 1), lambda i, k, b: (i, 0)),
        ],
        out_specs=pl.BlockSpec((TB, CS), lambda i, k, b: (i, 0)),
    )
    return pl.pallas_call(
        _ffn_body,
        grid_spec=grid_spec,
        out_shape=jax.ShapeDtypeStruct((2 * NSLOT, CS), jnp.float32),
    )(beid, xs, Wg, Wu, Wd, ws)


def _unsort(hs, t2s):
    mesh = plsc.VectorSubcoreMesh(core_axis_name="c", subcore_axis_name="s")

    @functools.partial(
        pl.kernel,
        mesh=mesh,
        compiler_params=pltpu.CompilerParams(needs_layout_passes=False),
        out_type=jax.ShapeDtypeStruct((2 * NTOK, CS), jnp.float32),
        scratch_types=[
            pltpu.VMEM((16,), jnp.int32),
            pltpu.VMEM((16, CS), jnp.float32),
            pltpu.SemaphoreType.DMA,
        ],
    )
    def k(hs_hbm, t2s_hbm, f_hbm, idx_v, rows_v, sem):
        w = lax.axis_index("s") * 2 + lax.axis_index("c")
        base = w * 128
        for j in range(8):
            pltpu.sync_copy(t2s_hbm.at[pl.ds(base + 16 * j, 16)], idx_v)
            pltpu.async_copy(hs_hbm.at[idx_v], rows_v, sem).wait()
            pltpu.sync_copy(rows_v, f_hbm.at[pl.ds(base + 16 * j, 16)])

    return k(hs, t2s)


def _wc_body(f0_ref, f1_ref, wc_ref, out_ref):
    wc = wc_ref[...]
    out_ref[...] = (
        jax.lax.dot_general(f0_ref[...], wc[:, :CS], (((1,), (1,)), ((), ())),
                            preferred_element_type=jnp.float32)
        + jax.lax.dot_general(f1_ref[...], wc[:, CS:], (((1,), (1,)), ((), ())),
                              preferred_element_type=jnp.float32)
    )


def _wc_matmul(F, Wc):
    tb = 256
    return pl.pallas_call(
        _wc_body,
        grid=(NTOK // tb,),
        in_specs=[
            pl.BlockSpec((tb, CS), lambda i: (i, 0)),
            pl.BlockSpec((tb, CS), lambda i: (i + NTOK // tb, 0)),
            pl.BlockSpec((D, D), lambda i: (0, 0)),
        ],
        out_specs=pl.BlockSpec((tb, D), lambda i: (i, 0)),
        out_shape=jax.ShapeDtypeStruct((NTOK, D), jnp.float32),
    )(F, F, Wc)


def kernel(hidden_states, Wgate, Wg, Wu, Wd, Wc):
    b, s_len, d = hidden_states.shape
    x = hidden_states.reshape(b * s_len, d)
    logits, ep, w0, w1 = _router(x, Wgate)
    wcat = jnp.concatenate([w0.reshape(-1), w1.reshape(-1)])
    xs, t2s, wsrt, beid = _sort_gather(x, ep.reshape(-1), wcat)
    hs = _ffn_grouped(beid, xs, Wg, Wu, Wd, wsrt.reshape(2 * NSLOT, 1))
    F = _unsort(hs, t2s)
    out = _wc_matmul(F, Wc)
    return out.reshape(b, s_len, d), logits


# speedup vs baseline: 1.4531x; 1.4531x over previous
import functools

import jax
import jax.numpy as jnp
from jax import lax
from jax.experimental import pallas as pl
from jax.experimental.pallas import tpu as pltpu
from jax.experimental.pallas import tpu_sc as plsc

E = 16
D = 2048
D_FF = 1024
NTOK = 2048
TB = 256
NBLK = 16
NSLOT = NBLK * TB
CS = D // 2


def _bc16(s):
    return jnp.full((16,), s, jnp.int32)


def _router_body(x_ref, wg_ref, logits_ref, ep_ref, w0_ref, w1_ref):
    x = x_ref[...]
    logits = jax.lax.dot_general(
        x, wg_ref[...], (((1,), (1,)), ((), ())),
        preferred_element_type=jnp.float32,
    )
    logits_ref[...] = logits

    tb = logits.shape[0]
    lane = jax.lax.broadcasted_iota(jnp.int32, (tb, E), 1)
    neg = jnp.float32(-jnp.inf)

    gsum, gmax, garg = [], [], []
    for g in range(4):
        mask = (lane // 4) == g
        gsum.append(jnp.sum(jnp.where(mask, logits, 0.0), axis=1, keepdims=True))
        mg = jnp.max(jnp.where(mask, logits, neg), axis=1, keepdims=True)
        gmax.append(mg)
        ag = jnp.min(
            jnp.where(mask & (logits == mg), lane, 99), axis=1, keepdims=True
        ) - 4 * g
        garg.append(ag)

    ew, probs = [], []
    for s in range(2):
        pick = gsum[2 * s + 1] > gsum[2 * s]
        e_s = jnp.where(pick, 4 + garg[2 * s + 1], garg[2 * s])
        p_s = jnp.where(pick, gmax[2 * s + 1], gmax[2 * s])
        ew.append(e_s)
        probs.append(p_s)

    m = jnp.maximum(probs[0], probs[1])
    z0 = jnp.exp(probs[0] - m)
    z1 = jnp.exp(probs[1] - m)
    w0_ref[...] = z0 / (z0 + z1)
    w1_ref[...] = z1 / (z0 + z1)
    ep_ref[...] = ew[0] + 8 * ew[1]


def _router(x, Wgate):
    tb = 256
    return pl.pallas_call(
        _router_body,
        grid=(NTOK // tb,),
        in_specs=[
            pl.BlockSpec((tb, D), lambda i: (i, 0)),
            pl.BlockSpec((E, D), lambda i: (0, 0)),
        ],
        out_specs=[
            pl.BlockSpec((tb, E), lambda i: (i, 0)),
            pl.BlockSpec((tb, 1), lambda i: (i, 0)),
            pl.BlockSpec((tb, 1), lambda i: (i, 0)),
            pl.BlockSpec((tb, 1), lambda i: (i, 0)),
        ],
        out_shape=[
            jax.ShapeDtypeStruct((NTOK, E), jnp.float32),
            jax.ShapeDtypeStruct((NTOK, 1), jnp.int32),
            jax.ShapeDtypeStruct((NTOK, 1), jnp.float32),
            jax.ShapeDtypeStruct((NTOK, 1), jnp.float32),
        ],
    )(x, Wgate)


def _sort(ep, wcat):
    mesh = plsc.VectorSubcoreMesh(core_axis_name="c", subcore_axis_name="s")

    @functools.partial(
        pl.kernel,
        mesh=mesh,
        compiler_params=pltpu.CompilerParams(needs_layout_passes=False),
        out_type=[
            jax.ShapeDtypeStruct((2 * NSLOT,), jnp.int32),
            jax.ShapeDtypeStruct((2 * NTOK,), jnp.int32),
            jax.ShapeDtypeStruct((2 * NSLOT,), jnp.float32),
            jax.ShapeDtypeStruct((2 * NBLK,), jnp.int32),
        ],
        scratch_types=[
            pltpu.VMEM((128,), jnp.int32),
            pltpu.VMEM((128,), jnp.float32),
            pltpu.VMEM((128,), jnp.int32),
            pltpu.VMEM((128,), jnp.int32),
            pltpu.VMEM((128,), jnp.int32),
            pltpu.VMEM((16,), jnp.int32),
            pltpu.VMEM((16, 16), jnp.int32),
            pltpu.VMEM((16,), jnp.int32),
            pltpu.VMEM((16,), jnp.int32),
            pltpu.VMEM((16,), jnp.int32),
            pltpu.VMEM((256,), jnp.int32),
            pltpu.VMEM((256,), jnp.float32),
            pltpu.VMEM_SHARED((16, 16), jnp.int32),
            pltpu.VMEM_SHARED((NSLOT,), jnp.int32),
            pltpu.VMEM_SHARED((NSLOT,), jnp.float32),
        ],
    )
    def k(ep_hbm, wcat_hbm, sidx_hbm, t2s_hbm, wsrt_hbm, beid_hbm,
          e_v, w_v, rank_v, slots_v, g_v, counts_v, allc_v, base_v, bo_v,
          beid_v, idx_v, wbuf_v, counts_sh, sidx_sh, wsrt_sh):
        c = lax.axis_index("c")
        t = lax.axis_index("s")
        tok0 = t * 128
        iota = lax.iota(jnp.int32, 16)

        pltpu.sync_copy(ep_hbm.at[pl.ds(tok0, 128)], e_v)
        pltpu.sync_copy(wcat_hbm.at[pl.ds(c * NTOK + tok0, 128)], w_v)
        c_is_1 = _bc16(c) == 1
        for i in range(8):
            v = e_v[pl.ds(16 * i, 16)]
            e_v[pl.ds(16 * i, 16)] = jnp.where(c_is_1, (v >> 3) & 7, v & 7)

        counts_v[...] = jnp.zeros((16,), jnp.int32)
        for i in range(8):
            ev = e_v[pl.ds(16 * i, 16)]
            prefix = plsc.load_gather(counts_v, [ev])
            within = jnp.zeros((16,), jnp.int32)
            newc = counts_v[...]
            for e in range(8):
                m = ev == e
                mi = m.astype(jnp.int32)
                cum = jnp.cumsum(mi)
                within = jnp.where(m, cum - 1, within)
                newc = newc + jnp.where(iota == e, _bc16(jnp.sum(mi)), 0)
            counts_v[...] = newc
            rank_v[pl.ds(16 * i, 16)] = prefix + within

        pltpu.sync_copy(counts_v, counts_sh.at[t])
        plsc.subcore_barrier()
        pltpu.sync_copy(counts_sh, allc_v)
        ctot = allc_v[0]
        for i in range(1, 16):
            ctot = ctot + allc_v[i]
        pre = jnp.zeros((16,), jnp.int32)
        tvec = _bc16(t)
        for i in range(16):
            pre = pre + jnp.where(tvec > i, allc_v[i], 0)
        padded = ((ctot + (TB - 1)) >> 8) << 8
        poff = jnp.cumsum(padded) - padded
        base_v[...] = poff + pre
        bo_v[...] = poff >> 8

        @pl.when(t == 0)
        def _():
            beid = jnp.zeros((16,), jnp.int32)
            for e in range(1, 8):
                boe = plsc.load_gather(bo_v, [_bc16(e)])
                beid = beid + jnp.where(iota >= boe, 1, 0)
            beid_v[...] = beid + _bc16(8 * c)
            pltpu.sync_copy(beid_v, beid_hbm.at[pl.ds(NBLK * c, NBLK)])

        for i in range(8):
            ev = e_v[pl.ds(16 * i, 16)]
            slots = plsc.load_gather(base_v, [ev]) + rank_v[pl.ds(16 * i, 16)]
            slots_v[pl.ds(16 * i, 16)] = slots
            g_v[pl.ds(16 * i, 16)] = slots + _bc16(c * NSLOT)
            e_v[pl.ds(16 * i, 16)] = iota + _bc16(tok0 + 16 * i)
        pltpu.sync_copy(g_v, t2s_hbm.at[pl.ds(c * NTOK + tok0, 128)])

        for j in range(16):
            idx_v[pl.ds(16 * j, 16)] = jnp.zeros((16,), jnp.int32)
        pltpu.sync_copy(idx_v, sidx_sh.at[pl.ds(256 * t, 256)])
        plsc.subcore_barrier()
        pltpu.sync_copy(e_v, sidx_sh.at[slots_v])
        pltpu.sync_copy(w_v, wsrt_sh.at[slots_v])
        plsc.subcore_barrier()

        pltpu.sync_copy(sidx_sh.at[pl.ds(256 * t, 256)], idx_v)
        pltpu.sync_copy(idx_v, sidx_hbm.at[pl.ds(c * NSLOT + 256 * t, 256)])
        pltpu.sync_copy(wsrt_sh.at[pl.ds(256 * t, 256)], wbuf_v)
        pltpu.sync_copy(wbuf_v, wsrt_hbm.at[pl.ds(c * NSLOT + 256 * t, 256)])

    return k(ep, wcat)


def _ffn_body(beid_ref, sidx_ref, x_ref, wgc_ref, wuc_ref, wdc_ref, ws_ref,
              out_ref, xb_ref):
    kf = pl.program_id(1)

    @pl.when(kf == 0)
    def _():
        sel = jax.lax.broadcasted_iota(jnp.int32, (TB, NTOK), 1) == sidx_ref[...]
        p = jnp.where(sel, 1.0, 0.0)
        xb_ref[...] = jax.lax.dot_general(
            p, x_ref[...], (((1,), (0,)), ((), ())),
            preferred_element_type=jnp.float32)
        out_ref[...] = jnp.zeros_like(out_ref)

    x = xb_ref[...]
    g = jax.lax.dot_general(x, wgc_ref[0], (((1,), (1,)), ((), ())),
                            preferred_element_type=jnp.float32)
    u = jax.lax.dot_general(x, wuc_ref[0], (((1,), (1,)), ((), ())),
                            preferred_element_type=jnp.float32)
    hc = (g * jax.nn.sigmoid(g)) * u
    z = jax.lax.dot_general(hc, wdc_ref[0], (((1,), (1,)), ((), ())),
                            preferred_element_type=jnp.float32)
    out_ref[...] += ws_ref[...] * z


def _ffn_grouped(beid, sidx, x, Wg, Wu, Wd, ws):
    ffc = 512
    grid_spec = pltpu.PrefetchScalarGridSpec(
        num_scalar_prefetch=1,
        grid=(2 * NBLK, D_FF // ffc),
        in_specs=[
            pl.BlockSpec((TB, 1), lambda i, k, b: (i, 0)),
            pl.BlockSpec((NTOK, D), lambda i, k, b: (0, 0)),
            pl.BlockSpec((1, ffc, D), lambda i, k, b: (b[i], k, 0)),
            pl.BlockSpec((1, ffc, D), lambda i, k, b: (b[i], k, 0)),
            pl.BlockSpec((1, CS, ffc), lambda i, k, b: (b[i], 0, k)),
            pl.BlockSpec((TB, 1), lambda i, k, b: (i, 0)),
        ],
        out_specs=pl.BlockSpec((TB, CS), lambda i, k, b: (i, 0)),
        scratch_shapes=[pltpu.VMEM((TB, D), jnp.float32)],
    )
    return pl.pallas_call(
        _ffn_body,
        grid_spec=grid_spec,
        out_shape=jax.ShapeDtypeStruct((2 * NSLOT, CS), jnp.float32),
    )(beid, sidx, x, Wg, Wu, Wd, ws)


def _wc_body(t0_ref, t1_ref, hs_ref, wc_ref, out_ref):
    tb = t0_ref.shape[0]
    lane = jax.lax.broadcasted_iota(jnp.int32, (tb, NSLOT), 1)
    u0 = jnp.where(lane == t0_ref[...], 1.0, 0.0)
    u1 = jnp.where(lane == (t1_ref[...] - NSLOT), 1.0, 0.0)
    f0 = jax.lax.dot_general(u0, hs_ref[:NSLOT], (((1,), (0,)), ((), ())),
                             preferred_element_type=jnp.float32)
    f1 = jax.lax.dot_general(u1, hs_ref[NSLOT:], (((1,), (0,)), ((), ())),
                             preferred_element_type=jnp.float32)
    wc = wc_ref[...]
    out_ref[...] = (
        jax.lax.dot_general(f0, wc[:, :CS], (((1,), (1,)), ((), ())),
                            preferred_element_type=jnp.float32)
        + jax.lax.dot_general(f1, wc[:, CS:], (((1,), (1,)), ((), ())),
                              preferred_element_type=jnp.float32)
    )


def _wc_matmul(t2s, hs, Wc):
    tb = 256
    nb = NTOK // tb
    return pl.pallas_call(
        _wc_body,
        grid=(nb,),
        in_specs=[
            pl.BlockSpec((tb, 1), lambda i: (i, 0)),
            pl.BlockSpec((tb, 1), lambda i: (i + nb, 0)),
            pl.BlockSpec((2 * NSLOT, CS), lambda i: (0, 0)),
            pl.BlockSpec((D, D), lambda i: (0, 0)),
        ],
        out_specs=pl.BlockSpec((tb, D), lambda i: (i, 0)),
        out_shape=jax.ShapeDtypeStruct((NTOK, D), jnp.float32),
    )(t2s, t2s, hs, Wc)


def kernel(hidden_states, Wgate, Wg, Wu, Wd, Wc):
    b, s_len, d = hidden_states.shape
    x = hidden_states.reshape(b * s_len, d)
    logits, ep, w0, w1 = _router(x, Wgate)
    wcat = jnp.concatenate([w0.reshape(-1), w1.reshape(-1)])
    sidx, t2s, wsrt, beid = _sort(ep.reshape(-1), wcat)
    hs = _ffn_grouped(beid, sidx.reshape(2 * NSLOT, 1), x, Wg, Wu, Wd,
                      wsrt.reshape(2 * NSLOT, 1))
    out = _wc_matmul(t2s.reshape(2 * NTOK, 1), hs, Wc)
    return out.reshape(b, s_len, d), logits


# speedup vs baseline: 1.5622x; 1.0751x over previous
import functools

import jax
import jax.numpy as jnp
from jax import lax
from jax.experimental import pallas as pl
from jax.experimental.pallas import tpu as pltpu
from jax.experimental.pallas import tpu_sc as plsc

E = 16
D = 2048
D_FF = 1024
NTOK = 2048
TB = 256
NBLK = 16
NSLOT = NBLK * TB
CS = D // 2


def _bc16(s):
    return jnp.full((16,), s, jnp.int32)


def _router_body(x_ref, wg_ref, logits_ref, ep_ref, w0_ref, w1_ref):
    x = x_ref[...]
    logits = jax.lax.dot_general(
        x, wg_ref[...], (((1,), (1,)), ((), ())),
        preferred_element_type=jnp.float32,
    )
    logits_ref[...] = logits

    tb = logits.shape[0]
    lane = jax.lax.broadcasted_iota(jnp.int32, (tb, E), 1)
    neg = jnp.float32(-jnp.inf)

    gsum, gmax, garg = [], [], []
    for g in range(4):
        mask = (lane // 4) == g
        gsum.append(jnp.sum(jnp.where(mask, logits, 0.0), axis=1, keepdims=True))
        mg = jnp.max(jnp.where(mask, logits, neg), axis=1, keepdims=True)
        gmax.append(mg)
        ag = jnp.min(
            jnp.where(mask & (logits == mg), lane, 99), axis=1, keepdims=True
        ) - 4 * g
        garg.append(ag)

    ew, probs = [], []
    for s in range(2):
        pick = gsum[2 * s + 1] > gsum[2 * s]
        e_s = jnp.where(pick, 4 + garg[2 * s + 1], garg[2 * s])
        p_s = jnp.where(pick, gmax[2 * s + 1], gmax[2 * s])
        ew.append(e_s)
        probs.append(p_s)

    m = jnp.maximum(probs[0], probs[1])
    z0 = jnp.exp(probs[0] - m)
    z1 = jnp.exp(probs[1] - m)
    w0_ref[...] = z0 / (z0 + z1)
    w1_ref[...] = z1 / (z0 + z1)
    ep_ref[...] = ew[0] + 8 * ew[1]


def _router(x, Wgate):
    tb = 256
    return pl.pallas_call(
        _router_body,
        grid=(NTOK // tb,),
        in_specs=[
            pl.BlockSpec((tb, D), lambda i: (i, 0)),
            pl.BlockSpec((E, D), lambda i: (0, 0)),
        ],
        out_specs=[
            pl.BlockSpec((tb, E), lambda i: (i, 0)),
            pl.BlockSpec((tb, 1), lambda i: (i, 0)),
            pl.BlockSpec((tb, 1), lambda i: (i, 0)),
            pl.BlockSpec((tb, 1), lambda i: (i, 0)),
        ],
        out_shape=[
            jax.ShapeDtypeStruct((NTOK, E), jnp.float32),
            jax.ShapeDtypeStruct((NTOK, 1), jnp.int32),
            jax.ShapeDtypeStruct((NTOK, 1), jnp.float32),
            jax.ShapeDtypeStruct((NTOK, 1), jnp.float32),
        ],
    )(x, Wgate)


def _sort(ep, wcat):
    mesh = plsc.VectorSubcoreMesh(core_axis_name="c", subcore_axis_name="s")

    @functools.partial(
        pl.kernel,
        mesh=mesh,
        compiler_params=pltpu.CompilerParams(needs_layout_passes=False),
        out_type=[
            jax.ShapeDtypeStruct((2 * NSLOT,), jnp.int32),
            jax.ShapeDtypeStruct((2 * NTOK,), jnp.int32),
            jax.ShapeDtypeStruct((2 * NSLOT,), jnp.float32),
            jax.ShapeDtypeStruct((2 * NBLK,), jnp.int32),
            jax.ShapeDtypeStruct((2 * NBLK,), jnp.int32),
        ],
        scratch_types=[
            pltpu.VMEM((128,), jnp.int32),
            pltpu.VMEM((128,), jnp.float32),
            pltpu.VMEM((128,), jnp.int32),
            pltpu.VMEM((128,), jnp.int32),
            pltpu.VMEM((128,), jnp.int32),
            pltpu.VMEM((16,), jnp.int32),
            pltpu.VMEM((16, 16), jnp.int32),
            pltpu.VMEM((16,), jnp.int32),
            pltpu.VMEM((16,), jnp.int32),
            pltpu.VMEM((16,), jnp.int32),
            pltpu.VMEM((16,), jnp.int32),
            pltpu.VMEM((256,), jnp.int32),
            pltpu.VMEM((256,), jnp.float32),
            pltpu.VMEM_SHARED((16, 16), jnp.int32),
            pltpu.VMEM_SHARED((NSLOT,), jnp.int32),
            pltpu.VMEM_SHARED((NSLOT,), jnp.float32),
        ],
    )
    def k(ep_hbm, wcat_hbm, sidx_hbm, t2s_hbm, wsrt_hbm, beid_hbm, valid_hbm,
          e_v, w_v, rank_v, slots_v, g_v, counts_v, allc_v, base_v, bo_v,
          beid_v, valid_v, idx_v, wbuf_v, counts_sh, sidx_sh, wsrt_sh):
        c = lax.axis_index("c")
        t = lax.axis_index("s")
        tok0 = t * 128
        iota = lax.iota(jnp.int32, 16)

        pltpu.sync_copy(ep_hbm.at[pl.ds(tok0, 128)], e_v)
        pltpu.sync_copy(wcat_hbm.at[pl.ds(c * NTOK + tok0, 128)], w_v)
        c_is_1 = _bc16(c) == 1
        for i in range(8):
            v = e_v[pl.ds(16 * i, 16)]
            e_v[pl.ds(16 * i, 16)] = jnp.where(c_is_1, (v >> 3) & 7, v & 7)

        counts_v[...] = jnp.zeros((16,), jnp.int32)
        for i in range(8):
            ev = e_v[pl.ds(16 * i, 16)]
            prefix = plsc.load_gather(counts_v, [ev])
            within = jnp.zeros((16,), jnp.int32)
            newc = counts_v[...]
            for e in range(8):
                m = ev == e
                mi = m.astype(jnp.int32)
                cum = jnp.cumsum(mi)
                within = jnp.where(m, cum - 1, within)
                newc = newc + jnp.where(iota == e, _bc16(jnp.sum(mi)), 0)
            counts_v[...] = newc
            rank_v[pl.ds(16 * i, 16)] = prefix + within

        pltpu.sync_copy(counts_v, counts_sh.at[t])
        plsc.subcore_barrier()
        pltpu.sync_copy(counts_sh, allc_v)
        ctot = allc_v[0]
        for i in range(1, 16):
            ctot = ctot + allc_v[i]
        pre = jnp.zeros((16,), jnp.int32)
        tvec = _bc16(t)
        for i in range(16):
            pre = pre + jnp.where(tvec > i, allc_v[i], 0)
        padded = ((ctot + (TB - 1)) >> 8) << 8
        poff = jnp.cumsum(padded) - padded
        base_v[...] = poff + pre
        bo_v[...] = poff >> 8

        @pl.when(t == 0)
        def _():
            nb = jnp.sum(padded) >> 8
            bi = jnp.minimum(iota, _bc16(nb - 1))
            beid = jnp.zeros((16,), jnp.int32)
            for e in range(1, 8):
                boe = plsc.load_gather(bo_v, [_bc16(e)])
                beid = beid + jnp.where(boe <= bi, 1, 0)
            beid_v[...] = beid + _bc16(8 * c)
            valid_v[...] = (iota < _bc16(nb)).astype(jnp.int32)
            pltpu.sync_copy(beid_v, beid_hbm.at[pl.ds(NBLK * c, NBLK)])
            pltpu.sync_copy(valid_v, valid_hbm.at[pl.ds(NBLK * c, NBLK)])

        for i in range(8):
            ev = e_v[pl.ds(16 * i, 16)]
            slots = plsc.load_gather(base_v, [ev]) + rank_v[pl.ds(16 * i, 16)]
            slots_v[pl.ds(16 * i, 16)] = slots
            g_v[pl.ds(16 * i, 16)] = slots + _bc16(c * NSLOT)
            e_v[pl.ds(16 * i, 16)] = iota + _bc16(tok0 + 16 * i)
        pltpu.sync_copy(g_v, t2s_hbm.at[pl.ds(c * NTOK + tok0, 128)])

        for j in range(16):
            idx_v[pl.ds(16 * j, 16)] = jnp.zeros((16,), jnp.int32)
        pltpu.sync_copy(idx_v, sidx_sh.at[pl.ds(256 * t, 256)])
        plsc.subcore_barrier()
        pltpu.sync_copy(e_v, sidx_sh.at[slots_v])
        pltpu.sync_copy(w_v, wsrt_sh.at[slots_v])
        plsc.subcore_barrier()

        pltpu.sync_copy(sidx_sh.at[pl.ds(256 * t, 256)], idx_v)
        pltpu.sync_copy(idx_v, sidx_hbm.at[pl.ds(c * NSLOT + 256 * t, 256)])
        pltpu.sync_copy(wsrt_sh.at[pl.ds(256 * t, 256)], wbuf_v)
        pltpu.sync_copy(wbuf_v, wsrt_hbm.at[pl.ds(c * NSLOT + 256 * t, 256)])

    return k(ep, wcat)


def _ffn_body(beid_ref, valid_ref, sidx_ref, x_ref, wgc_ref, wuc_ref, wdc_ref,
              ws_ref, out_ref, xb_ref, acc_ref):
    i = pl.program_id(0)
    kf = pl.program_id(1)

    @pl.when(valid_ref[i] > 0)
    def _():
        @pl.when(kf == 0)
        def _():
            sel = (jax.lax.broadcasted_iota(jnp.int32, (TB, NTOK), 1)
                   == sidx_ref[...])
            p = jnp.where(sel, 1.0, 0.0)
            xb_ref[...] = jax.lax.dot_general(
                p, x_ref[...], (((1,), (0,)), ((), ())),
                preferred_element_type=jnp.float32)

        x = xb_ref[...]
        g = jax.lax.dot_general(x, wgc_ref[0], (((1,), (1,)), ((), ())),
                                preferred_element_type=jnp.float32)
        u = jax.lax.dot_general(x, wuc_ref[0], (((1,), (1,)), ((), ())),
                                preferred_element_type=jnp.float32)
        hc = (g * jax.nn.sigmoid(g)) * u
        z = jax.lax.dot_general(hc, wdc_ref[0], (((1,), (1,)), ((), ())),
                                preferred_element_type=jnp.float32)
        zw = ws_ref[...] * z

        @pl.when(kf == 0)
        def _():
            acc_ref[...] = zw

        @pl.when(kf == 1)
        def _():
            out_ref[...] = (acc_ref[...] + zw).astype(jnp.bfloat16)


def _ffn_grouped(beid, valid, sidx, x, Wg, Wu, Wd, ws):
    ffc = 512
    grid_spec = pltpu.PrefetchScalarGridSpec(
        num_scalar_prefetch=2,
        grid=(2 * NBLK, D_FF // ffc),
        in_specs=[
            pl.BlockSpec((TB, 1), lambda i, k, b, v: (i, 0)),
            pl.BlockSpec((NTOK, D), lambda i, k, b, v: (0, 0)),
            pl.BlockSpec((1, ffc, D), lambda i, k, b, v: (b[i], k, 0)),
            pl.BlockSpec((1, ffc, D), lambda i, k, b, v: (b[i], k, 0)),
            pl.BlockSpec((1, CS, ffc), lambda i, k, b, v: (b[i], 0, k)),
            pl.BlockSpec((TB, 1), lambda i, k, b, v: (i, 0)),
        ],
        out_specs=pl.BlockSpec((TB, CS), lambda i, k, b, v: (i, 0)),
        scratch_shapes=[
            pltpu.VMEM((TB, D), jnp.float32),
            pltpu.VMEM((TB, CS), jnp.float32),
        ],
    )
    return pl.pallas_call(
        _ffn_body,
        grid_spec=grid_spec,
        out_shape=jax.ShapeDtypeStruct((2 * NSLOT, CS), jnp.bfloat16),
    )(beid, valid, sidx, x, Wg, Wu, Wd, ws)


def _wc_body(t0_ref, t1_ref, hs_ref, wc_ref, out_ref):
    tb = t0_ref.shape[0]
    lane = jax.lax.broadcasted_iota(jnp.int32, (tb, NSLOT), 1)
    u0 = jnp.where(lane == t0_ref[...], 1.0, 0.0).astype(jnp.bfloat16)
    u1 = jnp.where(lane == (t1_ref[...] - NSLOT), 1.0, 0.0).astype(jnp.bfloat16)
    f0 = jax.lax.dot_general(u0, hs_ref[:NSLOT], (((1,), (0,)), ((), ())),
                             preferred_element_type=jnp.float32)
    f1 = jax.lax.dot_general(u1, hs_ref[NSLOT:], (((1,), (0,)), ((), ())),
                             preferred_element_type=jnp.float32)
    wc = wc_ref[...]
    out_ref[...] = (
        jax.lax.dot_general(f0, wc[:, :CS], (((1,), (1,)), ((), ())),
                            preferred_element_type=jnp.float32)
        + jax.lax.dot_general(f1, wc[:, CS:], (((1,), (1,)), ((), ())),
                              preferred_element_type=jnp.float32)
    )


def _wc_matmul(t2s, hs, Wc):
    tb = 256
    nb = NTOK // tb
    return pl.pallas_call(
        _wc_body,
        grid=(nb,),
        in_specs=[
            pl.BlockSpec((tb, 1), lambda i: (i, 0)),
            pl.BlockSpec((tb, 1), lambda i: (i + nb, 0)),
            pl.BlockSpec((2 * NSLOT, CS), lambda i: (0, 0)),
            pl.BlockSpec((D, D), lambda i: (0, 0)),
        ],
        out_specs=pl.BlockSpec((tb, D), lambda i: (i, 0)),
        out_shape=jax.ShapeDtypeStruct((NTOK, D), jnp.float32),
    )(t2s, t2s, hs, Wc)


def kernel(hidden_states, Wgate, Wg, Wu, Wd, Wc):
    b, s_len, d = hidden_states.shape
    x = hidden_states.reshape(b * s_len, d)
    logits, ep, w0, w1 = _router(x, Wgate)
    wcat = jnp.concatenate([w0.reshape(-1), w1.reshape(-1)])
    sidx, t2s, wsrt, beid, valid = _sort(ep.reshape(-1), wcat)
    hs = _ffn_grouped(beid, valid, sidx.reshape(2 * NSLOT, 1), x, Wg, Wu, Wd,
                      wsrt.reshape(2 * NSLOT, 1))
    out = _wc_matmul(t2s.reshape(2 * NTOK, 1), hs, Wc)
    return out.reshape(b, s_len, d), logits
